# baseline (speedup 1.0000x reference)
import jax
import jax.numpy as jnp
from jax import lax
from jax.experimental import pallas as pl
from jax.experimental.pallas import tpu as pltpu
from jax.experimental.pallas import tpu_sc as plsc

N = 10000
D = 128
H = 256
C = 40
CP = 48
NPAD = 10240
NC, NS, LANES = 2, 16, 16
NW = NC * NS
CHUNK = 128
RING = 4
RBLK = 1000


def _mlp_body(x_ref, w1_ref, b1_ref, w2_ref, b2_ref, h_ref):
    h1 = jnp.dot(x_ref[...], w1_ref[...], preferred_element_type=jnp.float32)
    h1 = jnp.maximum(h1 + b1_ref[...], 0.0)
    h2 = jnp.dot(h1, w2_ref[...],
                 preferred_element_type=jnp.float32) + b2_ref[...]
    h_ref[...] = jnp.concatenate(
        [h2, jnp.zeros((RBLK, CP - C), jnp.float32)], axis=1)


def _combine_body(h_ref, a0_ref, a1_ref, o_ref):
    s = h_ref[...] + a0_ref[0] + a1_ref[0]
    col = lax.broadcasted_iota(jnp.int32, (RBLK, CP), 1)
    valid = col < C
    masked = jnp.where(valid, s, -jnp.inf)
    m = jnp.max(masked, axis=1, keepdims=True)
    ex = jnp.where(valid, jnp.exp(s - m), 0.0)
    lse = jnp.log(jnp.sum(ex, axis=1, keepdims=True)) + m
    o_ref[...] = (s - lse)[:, :C]


def _make_prop(nhops, npc):
    slab = NPAD // NS
    ngrp = npc // RING
    gq, grem = divmod(ngrp, NW)
    maxc = (gq + 1) * RING

    def body(h_hbm, ei_hbm, att_hbm, out_hbm,
             src_v, att_v, dstrow_v, rows, acc_sh, gsem, ssem, dsem):
        cid = lax.axis_index("c")
        sid = lax.axis_index("s")
        wid = sid * NC + cid
        cnt = jnp.where(wid < grem, gq + 1, gq) * RING
        start = (wid * gq + jnp.minimum(wid, grem)) * RING
        off = maxc - cnt
        read0 = (start - off) * CHUNK

        for l in range(nhops):
            pltpu.sync_copy(ei_hbm.at[l, 0, pl.ds(read0, maxc * CHUNK)],
                            src_v.at[l])

        zeros16 = jnp.zeros((LANES,), jnp.float32)

        @pl.loop(0, CHUNK)
        def _zero(e):
            for cc in range(CP // LANES):
                rows[0][e, pl.ds(cc * LANES, LANES)] = zeros16

        slab0 = sid * slab
        for i in range(slab // CHUNK):
            pltpu.sync_copy(rows[0], acc_sh.at[pl.ds(slab0 + i * CHUNK, CHUNK)])
        plsc.subcore_barrier()

        def issue_g(l, jr, t):
            pltpu.async_copy(h_hbm.at[src_v.at[l, pl.ds(jr * CHUNK, CHUNK)]],
                             rows[t], gsem[t])

        def wait_g(l, jr, t):
            pltpu.make_async_copy(
                h_hbm.at[src_v.at[l, pl.ds(jr * CHUNK, CHUNK)]], rows[t],
                gsem[t]).wait()

        def issue_d(l, jr, t):
            pltpu.async_copy(ei_hbm.at[l, 1, pl.ds(read0 + jr * CHUNK, CHUNK)],
                             dstrow_v.at[t], dsem[t])

        def wait_d(l, jr, t):
            pltpu.make_async_copy(
                ei_hbm.at[l, 1, pl.ds(read0 + jr * CHUNK, CHUNK)],
                dstrow_v.at[t], dsem[t]).wait()

        def issue_s(t):
            pltpu.async_copy(rows[t], acc_sh.at[dstrow_v.at[t]], ssem[t],
                             add=True)

        def wait_s(t):
            pltpu.make_async_copy(rows[t], acc_sh.at[dstrow_v.at[t]],
                                  ssem[t]).wait()

        def scale(jr, t):
            @pl.loop(0, CHUNK // LANES, unroll=2)
            def _scale(e16):
                av = att_v[pl.ds(jr * CHUNK + e16 * LANES, LANES)]
                for l in range(LANES):
                    a = av[l]
                    e = e16 * LANES + l
                    for cc in range(CP // LANES):
                        sl = pl.ds(cc * LANES, LANES)
                        rows[t][e, sl] = rows[t][e, sl] * a

        for l in range(nhops):
            pltpu.sync_copy(att_hbm.at[l, pl.ds(read0, maxc * CHUNK)], att_v)
            for t in range(RING):
                issue_g(l, off + t, t)
                issue_d(l, off + t, t)

            @pl.loop(0, cnt, step=RING)
            def _edges(jj):
                for t in range(RING):
                    jr = off + jj + t
                    wait_g(l, jr, t)
                    scale(jr, t)
                    wait_d(l, jr, t)
                    issue_s(t)
                for t in range(RING):
                    jr = off + jj + t
                    wait_s(t)
                    jn = jj + t + RING

                    @pl.when(jn < cnt)
                    def _():
                        issue_g(l, off + jn, t)
                        issue_d(l, off + jn, t)

        plsc.subcore_barrier()
        pltpu.sync_copy(acc_sh.at[pl.ds(slab0, slab)],
                        out_hbm.at[cid, pl.ds(slab0, slab)])

    mesh = plsc.VectorSubcoreMesh(core_axis_name="c", subcore_axis_name="s")
    return pl.kernel(
        body,
        out_type=jax.ShapeDtypeStruct((NC, NPAD, CP), jnp.float32),
        mesh=mesh,
        compiler_params=pltpu.CompilerParams(use_tc_tiling_on_sc=False),
        scratch_types=[
            pltpu.VMEM((nhops, maxc * CHUNK), jnp.int32),
            pltpu.VMEM((maxc * CHUNK,), jnp.float32),
            pltpu.VMEM((RING, CHUNK), jnp.int32),
            [pltpu.VMEM((CHUNK, CP), jnp.float32) for _ in range(RING)],
            pltpu.VMEM_SHARED((NPAD, CP), jnp.float32),
            [pltpu.SemaphoreType.DMA for _ in range(RING)],
            [pltpu.SemaphoreType.DMA for _ in range(RING)],
            [pltpu.SemaphoreType.DMA for _ in range(RING)],
        ],
    )


def kernel(x, edge_index, hop_edge_index, hop_edge_att, W1, b1, W2, b2):
    h = pl.pallas_call(
        _mlp_body,
        grid=(N // RBLK,),
        in_specs=[
            pl.BlockSpec((RBLK, D), lambda i: (i, 0)),
            pl.BlockSpec((D, H), lambda i: (0, 0)),
            pl.BlockSpec((1, H), lambda i: (0, 0)),
            pl.BlockSpec((H, C), lambda i: (0, 0)),
            pl.BlockSpec((1, C), lambda i: (0, 0)),
        ],
        out_specs=pl.BlockSpec((RBLK, CP), lambda i: (i, 0)),
        out_shape=jax.ShapeDtypeStruct((N, CP), jnp.float32),
    )(x, W1, b1.reshape(1, H), W2, b2.reshape(1, C))

    nhops, _, e_per_hop = hop_edge_index.shape
    npc = e_per_hop // CHUNK
    acc = _make_prop(nhops, npc)(h, hop_edge_index, hop_edge_att)

    return pl.pallas_call(
        _combine_body,
        grid=(N // RBLK,),
        in_specs=[
            pl.BlockSpec((RBLK, CP), lambda i: (i, 0)),
            pl.BlockSpec((1, RBLK, CP), lambda i: (0, i, 0)),
            pl.BlockSpec((1, RBLK, CP), lambda i: (1, i, 0)),
        ],
        out_specs=pl.BlockSpec((RBLK, C), lambda i: (i, 0)),
        out_shape=jax.ShapeDtypeStruct((N, C), jnp.float32),
    )(h, acc, acc)


# speedup vs baseline: 2.1701x; 2.1701x over previous
import jax
import jax.numpy as jnp
from jax import lax
from jax.experimental import pallas as pl
from jax.experimental.pallas import tpu as pltpu
from jax.experimental.pallas import tpu_sc as plsc

N = 10000
D = 128
H = 256
C = 40
CP = 48
NPAD = 10240
NC, NS, LANES = 2, 16, 16
NW = NC * NS
CHUNK = 128
RING = 4
RBLK = 1000


def _mlp_body(x_ref, w1_ref, b1_ref, w2_ref, b2_ref, h_ref):
    h1 = jnp.dot(x_ref[...], w1_ref[...], preferred_element_type=jnp.float32)
    h1 = jnp.maximum(h1 + b1_ref[...], 0.0)
    h2 = jnp.dot(h1, w2_ref[...],
                 preferred_element_type=jnp.float32) + b2_ref[...]
    h_ref[...] = jnp.concatenate(
        [h2, jnp.zeros((RBLK, CP - C), jnp.float32)], axis=1)


def _combine_body(h_ref, a0_ref, a1_ref, o_ref):
    s = h_ref[...] + a0_ref[0] + a1_ref[0]
    col = lax.broadcasted_iota(jnp.int32, (RBLK, CP), 1)
    valid = col < C
    masked = jnp.where(valid, s, -jnp.inf)
    m = jnp.max(masked, axis=1, keepdims=True)
    ex = jnp.where(valid, jnp.exp(s - m), 0.0)
    lse = jnp.log(jnp.sum(ex, axis=1, keepdims=True)) + m
    o_ref[...] = (s - lse)[:, :C]


def _make_prop(nhops, npc):
    slab = NPAD // NS
    ngrp = npc // RING
    gq, grem = divmod(ngrp, NW)
    maxc = (gq + 1) * RING

    def body(h_hbm, ei_hbm, att_hbm, out_hbm,
             src_v, att_v, dstrow_v, rows, acc_sh, gsem, ssem, dsem):
        cid = lax.axis_index("c")
        sid = lax.axis_index("s")
        wid = sid * NC + cid
        cnt = jnp.where(wid < grem, gq + 1, gq) * RING
        start = (wid * gq + jnp.minimum(wid, grem)) * RING
        off = maxc - cnt
        read0 = (start - off) * CHUNK

        for l in range(nhops):
            pltpu.sync_copy(ei_hbm.at[l, 0, pl.ds(read0, maxc * CHUNK)],
                            src_v.at[l])

        zeros16 = jnp.zeros((LANES,), jnp.float32)

        @pl.loop(0, CHUNK)
        def _zero(e):
            for cc in range(CP // LANES):
                rows[0][e, pl.ds(cc * LANES, LANES)] = zeros16

        slab0 = sid * slab
        for i in range(slab // CHUNK):
            pltpu.sync_copy(rows[0], acc_sh.at[pl.ds(slab0 + i * CHUNK, CHUNK)])
        plsc.subcore_barrier()

        def issue_g(l, jr, t):
            pltpu.async_copy(h_hbm.at[src_v.at[l, pl.ds(jr * CHUNK, CHUNK)]],
                             rows[t], gsem[t])

        def wait_g(l, jr, t):
            pltpu.make_async_copy(
                h_hbm.at[src_v.at[l, pl.ds(jr * CHUNK, CHUNK)]], rows[t],
                gsem[t]).wait()

        def issue_d(l, jr, t):
            pltpu.async_copy(ei_hbm.at[l, 1, pl.ds(read0 + jr * CHUNK, CHUNK)],
                             dstrow_v.at[t], dsem[t])

        def wait_d(l, jr, t):
            pltpu.make_async_copy(
                ei_hbm.at[l, 1, pl.ds(read0 + jr * CHUNK, CHUNK)],
                dstrow_v.at[t], dsem[t]).wait()

        def issue_s(t):
            pltpu.async_copy(rows[t], acc_sh.at[dstrow_v.at[t]], ssem[t],
                             add=True)

        def wait_s(t):
            pltpu.make_async_copy(rows[t], acc_sh.at[dstrow_v.at[t]],
                                  ssem[t]).wait()

        def scale(jr, t):
            @pl.loop(0, CHUNK // LANES)
            def _scale(e16):
                av = att_v[pl.ds(jr * CHUNK + e16 * LANES, LANES)]
                for l in range(LANES):
                    a = av[l]
                    e = e16 * LANES + l
                    for cc in range(CP // LANES):
                        sl = pl.ds(cc * LANES, LANES)
                        rows[t][e, sl] = rows[t][e, sl] * a

        for l in range(nhops):
            pltpu.sync_copy(att_hbm.at[l, pl.ds(read0, maxc * CHUNK)], att_v)
            for t in range(RING):
                issue_g(l, off + t, t)
                issue_d(l, off + t, t)

            @pl.loop(0, cnt, step=RING)
            def _edges(jj):
                for t in range(RING):
                    jr = off + jj + t
                    wait_g(l, jr, t)
                    scale(jr, t)
                    wait_d(l, jr, t)
                    issue_s(t)
                for t in range(RING):
                    jr = off + jj + t
                    wait_s(t)
                    jn = jj + t + RING

                    @pl.when(jn < cnt)
                    def _():
                        issue_g(l, off + jn, t)
                        issue_d(l, off + jn, t)

        plsc.subcore_barrier()
        pltpu.sync_copy(acc_sh.at[pl.ds(slab0, slab)],
                        out_hbm.at[cid, pl.ds(slab0, slab)])

    mesh = plsc.VectorSubcoreMesh(core_axis_name="c", subcore_axis_name="s")
    return pl.kernel(
        body,
        out_type=jax.ShapeDtypeStruct((NC, NPAD, CP), jnp.float32),
        mesh=mesh,
        compiler_params=pltpu.CompilerParams(use_tc_tiling_on_sc=False),
        scratch_types=[
            pltpu.VMEM((nhops, maxc * CHUNK), jnp.int32),
            pltpu.VMEM((maxc * CHUNK,), jnp.float32),
            pltpu.VMEM((RING, CHUNK), jnp.int32),
            [pltpu.VMEM((CHUNK, CP), jnp.float32) for _ in range(RING)],
            pltpu.VMEM_SHARED((NPAD, CP), jnp.float32),
            [pltpu.SemaphoreType.DMA for _ in range(RING)],
            [pltpu.SemaphoreType.DMA for _ in range(RING)],
            [pltpu.SemaphoreType.DMA for _ in range(RING)],
        ],
    )


def kernel(x, edge_index, hop_edge_index, hop_edge_att, W1, b1, W2, b2):
    h = pl.pallas_call(
        _mlp_body,
        grid=(N // RBLK,),
        in_specs=[
            pl.BlockSpec((RBLK, D), lambda i: (i, 0)),
            pl.BlockSpec((D, H), lambda i: (0, 0)),
            pl.BlockSpec((1, H), lambda i: (0, 0)),
            pl.BlockSpec((H, C), lambda i: (0, 0)),
            pl.BlockSpec((1, C), lambda i: (0, 0)),
        ],
        out_specs=pl.BlockSpec((RBLK, CP), lambda i: (i, 0)),
        out_shape=jax.ShapeDtypeStruct((N, CP), jnp.float32),
    )(x, W1, b1.reshape(1, H), W2, b2.reshape(1, C))

    nhops, _, e_per_hop = hop_edge_index.shape
    npc = e_per_hop // CHUNK
    acc = _make_prop(nhops, npc)(h, hop_edge_index, hop_edge_att)

    return pl.pallas_call(
        _combine_body,
        grid=(N // RBLK,),
        in_specs=[
            pl.BlockSpec((RBLK, CP), lambda i: (i, 0)),
            pl.BlockSpec((1, RBLK, CP), lambda i: (0, i, 0)),
            pl.BlockSpec((1, RBLK, CP), lambda i: (1, i, 0)),
        ],
        out_specs=pl.BlockSpec((RBLK, C), lambda i: (i, 0)),
        out_shape=jax.ShapeDtypeStruct((N, C), jnp.float32),
    )(h, acc, acc)
